# baseline (speedup 1.0000x reference)
import functools

import jax
import jax.numpy as jnp
from jax import lax
from jax.experimental import pallas as pl
from jax.experimental.pallas import tpu as pltpu
from jax.experimental.pallas import tpu_sc as plsc

_V = 100000
_D = 64
_B = 1024
_K = 4

_NC = 2
_NS = 16
_NW = _NC * _NS
_EX_PER_W = _B // _NW
_IDX_PER_W = _EX_PER_W * _K

_LANES = 16


def _gather_mean_body(idx_hbm, emb_hbm, h_hbm, idx_v, rows_v, h_v, sem):
    wid = lax.axis_index("s") * _NC + lax.axis_index("c")
    base = wid * _IDX_PER_W
    pltpu.sync_copy(idx_hbm.at[pl.ds(base, _IDX_PER_W)], idx_v)
    pltpu.async_copy(emb_hbm.at[idx_v], rows_v, sem).wait()
    for i in range(_EX_PER_W):
        for c in range(_D // _LANES):
            sl = pl.ds(c * _LANES, _LANES)
            acc = (rows_v[_K * i, sl] + rows_v[_K * i + 1, sl]
                   + rows_v[_K * i + 2, sl] + rows_v[_K * i + 3, sl])
            h_v[i, sl] = acc * (1.0 / _K)
    pltpu.sync_copy(h_v, h_hbm.at[pl.ds(wid * _EX_PER_W, _EX_PER_W)])


_gather_mean = functools.partial(
    pl.kernel,
    mesh=plsc.VectorSubcoreMesh(core_axis_name="c", subcore_axis_name="s"),
    out_type=jax.ShapeDtypeStruct((_B, _D), jnp.float32),
    scratch_types=[
        pltpu.VMEM((_IDX_PER_W,), jnp.int32),
        pltpu.VMEM((_IDX_PER_W, _D), jnp.float32),
        pltpu.VMEM((_EX_PER_W, _D), jnp.float32),
        pltpu.SemaphoreType.DMA,
    ],
    compiler_params=pltpu.CompilerParams(use_tc_tiling_on_sc=False),
)(_gather_mean_body)


_VB = 4096


_BB = 8
_NBUF = 8


def _proj_body(h_ref, w_ref, b_ref, o_hbm, buf, sems):
    i = pl.program_id(0)
    n = pl.num_programs(0)
    k = lax.rem(i, _NBUF)

    @pl.when(i >= _NBUF)
    def _wait_prev():
        pltpu.make_async_copy(
            buf.at[k], o_hbm.at[pl.ds(i * _BB, _BB)], sems.at[k]).wait()

    buf[k] = jnp.broadcast_to(b_ref[...] + h_ref[0, 0] + w_ref[0, 0],
                              (_BB, _V))
    pltpu.make_async_copy(
        buf.at[k], o_hbm.at[pl.ds(i * _BB, _BB)], sems.at[k]).start()

    @pl.when(i == n - 1)
    def _drain():
        for j in range(_NBUF):
            pltpu.make_async_copy(
                buf.at[j], o_hbm.at[pl.ds(j * _BB, _BB)], sems.at[j]).wait()


def _project(h, w, b2):
    return pl.pallas_call(
        _proj_body,
        grid=(_B // _BB,),
        in_specs=[
            pl.BlockSpec((_BB, _D), lambda i: (i, 0)),
            pl.BlockSpec((8, 128), lambda i: (0, 0)),
            pl.BlockSpec((1, _V), lambda i: (0, 0)),
        ],
        out_specs=pl.BlockSpec(memory_space=pl.ANY),
        out_shape=jax.ShapeDtypeStruct((_B, _V), jnp.float32),
        scratch_shapes=[
            pltpu.VMEM((_NBUF, _BB, _V), jnp.float32),
            pltpu.SemaphoreType.DMA((_NBUF,)),
        ],
    )(h, w, b2)


def kernel(x, emb, W, b):
    h = jnp.mean(jnp.take(emb, x.reshape(-1, _K), axis=0), axis=1)
    return _project(h, W, b.reshape(1, _V))


# speedup vs baseline: 1.7627x; 1.7627x over previous
import functools

import jax
import jax.numpy as jnp
from jax import lax
from jax.experimental import pallas as pl
from jax.experimental.pallas import tpu as pltpu
from jax.experimental.pallas import tpu_sc as plsc

_V = 100000
_D = 64
_B = 1024
_K = 4

_NC = 2
_NS = 16
_NW = _NC * _NS
_EX_PER_W = _B // _NW
_IDX_PER_W = _EX_PER_W * _K

_LANES = 16


def _gather_mean_body(idx_hbm, emb_hbm, h_hbm, idx_v, rows_v, h_v, sem):
    wid = lax.axis_index("s") * _NC + lax.axis_index("c")
    base = wid * _IDX_PER_W
    pltpu.sync_copy(idx_hbm.at[pl.ds(base, _IDX_PER_W)], idx_v)
    pltpu.async_copy(emb_hbm.at[idx_v], rows_v, sem).wait()
    for i in range(_EX_PER_W):
        for c in range(_D // _LANES):
            sl = pl.ds(c * _LANES, _LANES)
            acc = (rows_v[_K * i, sl] + rows_v[_K * i + 1, sl]
                   + rows_v[_K * i + 2, sl] + rows_v[_K * i + 3, sl])
            h_v[i, sl] = acc * (1.0 / _K)
    pltpu.sync_copy(h_v, h_hbm.at[pl.ds(wid * _EX_PER_W, _EX_PER_W)])


_gather_mean = functools.partial(
    pl.kernel,
    mesh=plsc.VectorSubcoreMesh(core_axis_name="c", subcore_axis_name="s"),
    out_type=jax.ShapeDtypeStruct((_B, _D), jnp.float32),
    scratch_types=[
        pltpu.VMEM((_IDX_PER_W,), jnp.int32),
        pltpu.VMEM((_IDX_PER_W, _D), jnp.float32),
        pltpu.VMEM((_EX_PER_W, _D), jnp.float32),
        pltpu.SemaphoreType.DMA,
    ],
    compiler_params=pltpu.CompilerParams(use_tc_tiling_on_sc=False),
)(_gather_mean_body)


_VB = 2048


def _proj_body(w_ref, h_ref, b_ref, ot_ref):
    ot_ref[...] = lax.dot_general(
        w_ref[...], h_ref[...],
        dimension_numbers=(((1,), (1,)), ((), ())),
        preferred_element_type=jnp.float32,
    ) + b_ref[...]


def _project_t(w, h, bc):
    return pl.pallas_call(
        _proj_body,
        grid=(pl.cdiv(_V, _VB),),
        in_specs=[
            pl.BlockSpec((_VB, _D), lambda i: (i, 0)),
            pl.BlockSpec((_B, _D), lambda i: (0, 0)),
            pl.BlockSpec((_VB, 1), lambda i: (i, 0)),
        ],
        out_specs=pl.BlockSpec((_VB, _B), lambda i: (i, 0)),
        out_shape=jax.ShapeDtypeStruct((_V, _B), jnp.float32),
    )(w, h, bc)


def kernel(x, emb, W, b):
    idx = x.reshape(-1).astype(jnp.int32)
    h = _gather_mean(idx, emb)
    out_t = _project_t(W, h, b.reshape(_V, 1))
    return out_t.T


# speedup vs baseline: 2.1625x; 1.2268x over previous
import functools

import jax
import jax.numpy as jnp
from jax import lax
from jax.experimental import pallas as pl
from jax.experimental.pallas import tpu as pltpu
from jax.experimental.pallas import tpu_sc as plsc

_V = 100000
_D = 64
_B = 1024
_K = 4

_NC = 2
_NS = 16
_NW = _NC * _NS
_EX_PER_W = _B // _NW
_IDX_PER_W = _EX_PER_W * _K

_LANES = 16


def _gather_mean_body(idx_hbm, emb_hbm, h_hbm, idx_v, rows_v, h_v, sem):
    wid = lax.axis_index("s") * _NC + lax.axis_index("c")
    base = wid * _IDX_PER_W
    pltpu.sync_copy(idx_hbm.at[pl.ds(base, _IDX_PER_W)], idx_v)
    pltpu.async_copy(emb_hbm.at[idx_v], rows_v, sem).wait()
    for i in range(_EX_PER_W):
        for c in range(_D // _LANES):
            sl = pl.ds(c * _LANES, _LANES)
            acc = (rows_v[_K * i, sl] + rows_v[_K * i + 1, sl]
                   + rows_v[_K * i + 2, sl] + rows_v[_K * i + 3, sl])
            h_v[i, sl] = acc * (1.0 / _K)
    pltpu.sync_copy(h_v, h_hbm.at[pl.ds(wid * _EX_PER_W, _EX_PER_W)])


_gather_mean = functools.partial(
    pl.kernel,
    mesh=plsc.VectorSubcoreMesh(core_axis_name="c", subcore_axis_name="s"),
    out_type=jax.ShapeDtypeStruct((_B, _D), jnp.float32),
    scratch_types=[
        pltpu.VMEM((_IDX_PER_W,), jnp.int32),
        pltpu.VMEM((_IDX_PER_W, _D), jnp.float32),
        pltpu.VMEM((_EX_PER_W, _D), jnp.float32),
        pltpu.SemaphoreType.DMA,
    ],
    compiler_params=pltpu.CompilerParams(use_tc_tiling_on_sc=False),
)(_gather_mean_body)


_VB = 4096


def _proj_body(w_ref, h_ref, b_ref, ot_ref):
    bcol = jnp.transpose(b_ref[...])
    ot_ref[...] = lax.dot_general(
        w_ref[...], h_ref[...],
        dimension_numbers=(((1,), (1,)), ((), ())),
        preferred_element_type=jnp.float32,
    ) + bcol


def _project_t(w, h, br):
    return pl.pallas_call(
        _proj_body,
        grid=(pl.cdiv(_V, _VB),),
        in_specs=[
            pl.BlockSpec((_VB, _D), lambda i: (i, 0)),
            pl.BlockSpec((_B, _D), lambda i: (0, 0)),
            pl.BlockSpec((1, _VB), lambda i: (0, i)),
        ],
        out_specs=pl.BlockSpec((_VB, _B), lambda i: (i, 0)),
        out_shape=jax.ShapeDtypeStruct((_V, _B), jnp.float32),
    )(w, h, br)


def kernel(x, emb, W, b):
    idx = x.reshape(-1).astype(jnp.int32)
    h = _gather_mean(idx, emb)
    out_t = _project_t(W, h, b.reshape(1, _V))
    return out_t.T


# speedup vs baseline: 2.5395x; 1.1743x over previous
import functools

import jax
import jax.numpy as jnp
from jax import lax
from jax.experimental import pallas as pl
from jax.experimental.pallas import tpu as pltpu
from jax.experimental.pallas import tpu_sc as plsc

_V = 100000
_D = 64
_B = 1024
_K = 4

_NC = 2
_NS = 16
_NW = _NC * _NS
_EX_PER_W = _B // _NW
_IDX_PER_W = _EX_PER_W * _K

_LANES = 16


_HALF = _IDX_PER_W // 2


def _gather_mean_body(idx_hbm, emb_hbm, h_hbm, idx_v, tiles_v, h_v, sem):
    wid = lax.axis_index("s") * _NC + lax.axis_index("c")
    base = wid * _IDX_PER_W
    pltpu.sync_copy(idx_hbm.at[pl.ds(base, _IDX_PER_W)], idx_v)
    for half in range(2):
        rvecs = []
        for blk in range(_HALF // _LANES):
            ivec = idx_v[pl.ds(half * _HALF + blk * _LANES, _LANES)]
            tvec = lax.shift_right_logical(ivec, 3)
            rvecs.append(ivec & 7)
            for l in range(_LANES):
                g = blk * _LANES + l
                pltpu.async_copy(
                    emb_hbm.at[pl.ds(tvec[l], 1)],
                    tiles_v.at[pl.ds(g, 1)], sem)
        for blk in range(_HALF // _LANES):
            for l in range(_LANES):
                pltpu.make_async_copy(
                    emb_hbm.at[pl.ds(0, 1)],
                    tiles_v.at[pl.ds(blk * _LANES + l, 1)], sem).wait()
        for e in range(_HALF // _K):
            i = half * (_HALF // _K) + e
            for c in range(_D // _LANES):
                sl = pl.ds(c * _LANES, _LANES)
                acc = jnp.zeros((_LANES,), jnp.float32)
                for j in range(_K):
                    g = _K * e + j
                    r = rvecs[g // _LANES][g % _LANES]
                    acc = acc + tiles_v[g, r, sl]
                h_v[i, sl] = acc * (1.0 / _K)
    pltpu.sync_copy(h_v, h_hbm.at[pl.ds(wid * _EX_PER_W, _EX_PER_W)])


_gather_mean = functools.partial(
    pl.kernel,
    mesh=plsc.VectorSubcoreMesh(core_axis_name="c", subcore_axis_name="s"),
    out_type=jax.ShapeDtypeStruct((_B, _D), jnp.float32),
    scratch_types=[
        pltpu.VMEM((_IDX_PER_W,), jnp.int32),
        pltpu.VMEM((_HALF, 8, _D), jnp.float32),
        pltpu.VMEM((_EX_PER_W, _D), jnp.float32),
        pltpu.SemaphoreType.DMA,
    ],
)(_gather_mean_body)


_VB = 4096


def _proj_body(w_ref, h_ref, b_ref, ot_ref):
    bcol = jnp.transpose(b_ref[...])
    ot_ref[...] = lax.dot_general(
        w_ref[...], h_ref[...],
        dimension_numbers=(((1,), (1,)), ((), ())),
        preferred_element_type=jnp.float32,
    ) + bcol


def _project_t(w, h, br):
    return pl.pallas_call(
        _proj_body,
        grid=(pl.cdiv(_V, _VB),),
        in_specs=[
            pl.BlockSpec((_VB, _D), lambda i: (i, 0)),
            pl.BlockSpec((_B, _D), lambda i: (0, 0)),
            pl.BlockSpec((1, _VB), lambda i: (0, i)),
        ],
        out_specs=pl.BlockSpec((_VB, _B), lambda i: (i, 0)),
        out_shape=jax.ShapeDtypeStruct((_V, _B), jnp.float32),
    )(w, h, br)


def kernel(x, emb, W, b):
    idx = x.reshape(-1).astype(jnp.int32)
    h = _gather_mean(idx, emb.reshape(_V // 8, 8, _D))
    out_t = _project_t(W, h, b.reshape(1, _V))
    return out_t.T


# speedup vs baseline: 2.9487x; 1.1611x over previous
import functools

import jax
import jax.numpy as jnp
from jax import lax
from jax.experimental import pallas as pl
from jax.experimental.pallas import tpu as pltpu
from jax.experimental.pallas import tpu_sc as plsc

_V = 100000
_D = 64
_B = 1024
_K = 4

_NC = 2
_NS = 16
_NW = _NC * _NS
_EX_PER_W = _B // _NW
_IDX_PER_W = _EX_PER_W * _K

_LANES = 16


_HALF = _IDX_PER_W // 2


def _gather_mean_body(idx_hbm, emb_hbm, h_hbm, idx_v, tiles_v, h_v, sem):
    wid = lax.axis_index("s") * _NC + lax.axis_index("c")
    base = wid * _IDX_PER_W
    pltpu.sync_copy(idx_hbm.at[pl.ds(base, _IDX_PER_W)], idx_v)
    for half in range(2):
        rvecs = []
        for blk in range(_HALF // _LANES):
            ivec = idx_v[pl.ds(half * _HALF + blk * _LANES, _LANES)]
            tvec = lax.shift_right_logical(ivec, 3)
            rvecs.append(ivec & 7)
            for l in range(_LANES):
                g = blk * _LANES + l
                pltpu.async_copy(
                    emb_hbm.at[pl.ds(tvec[l], 1)],
                    tiles_v.at[pl.ds(g, 1)], sem)
        for blk in range(_HALF // _LANES):
            for l in range(_LANES):
                pltpu.make_async_copy(
                    emb_hbm.at[pl.ds(0, 1)],
                    tiles_v.at[pl.ds(blk * _LANES + l, 1)], sem).wait()
        for e in range(_HALF // _K):
            i = half * (_HALF // _K) + e
            for c in range(_D // _LANES):
                sl = pl.ds(c * _LANES, _LANES)
                acc = jnp.zeros((_LANES,), jnp.float32)
                for j in range(_K):
                    g = _K * e + j
                    r = rvecs[g // _LANES][g % _LANES]
                    acc = acc + tiles_v[g, r, sl]
                h_v[i, sl] = acc * (1.0 / _K)
    pltpu.sync_copy(h_v, h_hbm.at[pl.ds(wid * _EX_PER_W, _EX_PER_W)])


_gather_mean = functools.partial(
    pl.kernel,
    mesh=plsc.VectorSubcoreMesh(core_axis_name="c", subcore_axis_name="s"),
    out_type=jax.ShapeDtypeStruct((_B, _D), jnp.float32),
    scratch_types=[
        pltpu.VMEM((_IDX_PER_W,), jnp.int32),
        pltpu.VMEM((_HALF, 8, _D), jnp.float32),
        pltpu.VMEM((_EX_PER_W, _D), jnp.float32),
        pltpu.SemaphoreType.DMA,
    ],
)(_gather_mean_body)


_VB = 4096


def _proj_body(wt_ref, h_ref, b_ref, ot_ref):
    bcol = jnp.transpose(b_ref[...])
    ot_ref[...] = lax.dot_general(
        wt_ref[...], h_ref[...],
        dimension_numbers=(((0,), (1,)), ((), ())),
        preferred_element_type=jnp.float32,
    ) + bcol


def _project_t(wt, h, br):
    return pl.pallas_call(
        _proj_body,
        grid=(pl.cdiv(_V, _VB),),
        in_specs=[
            pl.BlockSpec((_D, _VB), lambda i: (0, i)),
            pl.BlockSpec((_B, _D), lambda i: (0, 0)),
            pl.BlockSpec((1, _VB), lambda i: (0, i)),
        ],
        out_specs=pl.BlockSpec((_VB, _B), lambda i: (i, 0)),
        out_shape=jax.ShapeDtypeStruct((_V, _B), jnp.float32),
    )(wt, h, br)


def kernel(x, emb, W, b):
    idx = x.reshape(-1).astype(jnp.int32)
    h = _gather_mean(idx, emb.reshape(_V // 8, 8, _D))
    out_t = _project_t(W.T, h, b.reshape(1, _V))
    return out_t.T
